# baseline (speedup 1.0000x reference)
import functools

import jax
import jax.numpy as jnp
from jax import lax
from jax.experimental import pallas as pl
from jax.experimental.pallas import tpu as pltpu
from jax.experimental.pallas import tpu_sc as plsc

V = 1000000
D = 64
B = 16384

NC, NS, L = 2, 16, 16
NW = NC * NS
BPW = B // NW
NBLK = BPW // L


def _body(wi_i_hbm, wi_j_hbm, wi_hbm, wj_hbm, bi_hbm, bj_hbm, out_hbm,
          idx_i, idx_j, rows_i, rows_j, bv_i, bv_j, out_v, sem):
    wid = lax.axis_index("s") * NC + lax.axis_index("c")
    base = wid * BPW

    pltpu.sync_copy(wi_i_hbm.at[pl.ds(base, BPW)], idx_i)
    pltpu.sync_copy(wi_j_hbm.at[pl.ds(base, BPW)], idx_j)

    c1 = pltpu.async_copy(wi_hbm.at[idx_i], rows_i, sem)
    c2 = pltpu.async_copy(wj_hbm.at[idx_j], rows_j, sem)
    c3 = pltpu.async_copy(bi_hbm.at[idx_i], bv_i, sem)
    c4 = pltpu.async_copy(bj_hbm.at[idx_j], bv_j, sem)
    c1.wait()
    c2.wait()
    c3.wait()
    c4.wait()

    iota = lax.iota(jnp.int32, L)

    def block(b, carry):
        r0 = b * L
        acc = jnp.zeros((L,), jnp.float32)
        for r in range(L):
            row = r0 + r
            ri = rows_i.at[row]
            rj = rows_j.at[row]
            s = ri[pl.ds(0, L)] * rj[pl.ds(0, L)]
            for c in range(1, D // L):
                s = s + ri[pl.ds(c * L, L)] * rj[pl.ds(c * L, L)]
            acc = jnp.where(iota == r, jnp.sum(s), acc)
        acc = acc + bv_i[pl.ds(r0, L)] + bv_j[pl.ds(r0, L)]
        out_v[pl.ds(r0, L)] = acc
        return carry

    lax.fori_loop(0, NBLK, block, 0, unroll=False)

    pltpu.sync_copy(out_v, out_hbm.at[pl.ds(base, BPW)])


@functools.partial(jax.jit, static_argnames=())
def kernel(word_i, word_j, wi, wj, bi, bj):
    mesh = plsc.VectorSubcoreMesh(core_axis_name="c", subcore_axis_name="s")
    k = pl.kernel(
        _body,
        out_type=jax.ShapeDtypeStruct((B,), jnp.float32),
        mesh=mesh,
        compiler_params=pltpu.CompilerParams(
            needs_layout_passes=False, use_tc_tiling_on_sc=False),
        scratch_types=[
            pltpu.VMEM((BPW,), jnp.int32),
            pltpu.VMEM((BPW,), jnp.int32),
            pltpu.VMEM((BPW, D), jnp.float32),
            pltpu.VMEM((BPW, D), jnp.float32),
            pltpu.VMEM((BPW,), jnp.float32),
            pltpu.VMEM((BPW,), jnp.float32),
            pltpu.VMEM((BPW,), jnp.float32),
            pltpu.SemaphoreType.DMA,
        ],
    )
    return k(word_i.astype(jnp.int32), word_j.astype(jnp.int32), wi, wj,
             bi.reshape(V), bj.reshape(V))
